# Initial kernel scaffold; baseline (speedup 1.0000x reference)
import jax
import jax.numpy as jnp
from jax.experimental import pallas as pl


def kernel(x, edge_index, batch, W1_l, W1_r, b1, W2_l, W2_r, b2, W3_l, W3_r, b3):
    raise NotImplementedError("write your pallas kernel here")


# speedup vs baseline: 3.6255x; 3.6255x over previous
import functools

import jax
import jax.numpy as jnp
from jax import lax
from jax.experimental import pallas as pl
from jax.experimental.pallas import tpu as pltpu
from jax.experimental.pallas import tpu_sc as plsc

N = 10000
NP = 10240
E = 160000
EP = 163840
B = 128
RT = NP // 16

_mesh = plsc.VectorSubcoreMesh(core_axis_name="c", subcore_axis_name="s")


def _sc_l1_body(x_hbm, src_hbm, dst_hbm, zacc_hbm, zdeg_hbm,
                agg_out, deg_out,
                srcv, dstv, rows, ones, acc, dacc, sem):
  c = lax.axis_index("c")
  s = lax.axis_index("s")
  r0 = s * RT
  pltpu.sync_copy(zacc_hbm.at[pl.ds(r0, RT), :], acc.at[pl.ds(r0, RT), :])
  pltpu.sync_copy(zdeg_hbm.at[pl.ds(r0, RT)], dacc.at[pl.ds(r0, RT)])
  for i in range(B // 16):
    ones[pl.ds(i * 16, 16)] = jnp.full((16,), 1.0, jnp.float32)
  plsc.subcore_barrier()
  e0 = (c * 16 + s) * (EP // 32)

  def body(j, carry):
    off = e0 + j * B
    pltpu.sync_copy(src_hbm.at[pl.ds(off, B)], srcv)
    pltpu.sync_copy(dst_hbm.at[pl.ds(off, B)], dstv)
    pltpu.async_copy(x_hbm.at[srcv], rows, sem).wait()
    pltpu.sync_copy(rows, acc.at[dstv], add=True)
    pltpu.sync_copy(ones, dacc.at[dstv], add=True)
    return carry

  lax.fori_loop(0, (EP // 32) // B, body, 0)
  plsc.subcore_barrier()
  pltpu.sync_copy(acc.at[pl.ds(r0, RT), :], agg_out.at[c, pl.ds(r0, RT), :])
  pltpu.sync_copy(dacc.at[pl.ds(r0, RT)], deg_out.at[c, pl.ds(r0, RT)])


_sc_l1 = functools.partial(
    pl.kernel,
    out_type=(jax.ShapeDtypeStruct((2, NP, 128), jnp.float32),
              jax.ShapeDtypeStruct((2, NP), jnp.float32)),
    mesh=_mesh,
    scratch_types=[
        pltpu.VMEM((B,), jnp.int32),
        pltpu.VMEM((B,), jnp.int32),
        pltpu.VMEM((B, 128), jnp.float32),
        pltpu.VMEM((B,), jnp.float32),
        pltpu.VMEM_SHARED((NP, 128), jnp.float32),
        pltpu.VMEM_SHARED((NP,), jnp.float32),
        pltpu.SemaphoreType.DMA,
    ],
)(_sc_l1_body)


def _sc_l2_body(h14_hbm, src_hbm, dst_hbm, zacc_hbm,
                agg_out,
                srcv, dstv, idxv, rows, acc, sem):
  c = lax.axis_index("c")
  s = lax.axis_index("s")
  r0 = s * RT
  ept = EP // 16
  e0 = s * ept
  for r in range(2):
    fb = c * 2 + r
    pltpu.sync_copy(zacc_hbm.at[pl.ds(r0, RT), :], acc.at[pl.ds(r0, RT), :])
    plsc.subcore_barrier()

    def body(j, carry):
      off = e0 + j * B
      pltpu.sync_copy(src_hbm.at[pl.ds(off, B)], srcv)
      pltpu.sync_copy(dst_hbm.at[pl.ds(off, B)], dstv)
      for i in range(B // 16):
        sl = pl.ds(i * 16, 16)
        idxv[sl] = srcv[sl] * 4 + fb
      pltpu.async_copy(h14_hbm.at[idxv], rows, sem).wait()
      pltpu.sync_copy(rows, acc.at[dstv], add=True)
      return carry

    lax.fori_loop(0, ept // B, body, 0)
    plsc.subcore_barrier()
    pltpu.sync_copy(acc.at[pl.ds(r0, RT), :], agg_out.at[fb, pl.ds(r0, RT), :])
    plsc.subcore_barrier()


_sc_l2 = functools.partial(
    pl.kernel,
    out_type=jax.ShapeDtypeStruct((4, NP, 128), jnp.float32),
    mesh=_mesh,
    scratch_types=[
        pltpu.VMEM((B,), jnp.int32),
        pltpu.VMEM((B,), jnp.int32),
        pltpu.VMEM((B,), jnp.int32),
        pltpu.VMEM((B, 128), jnp.float32),
        pltpu.VMEM_SHARED((NP, 128), jnp.float32),
        pltpu.SemaphoreType.DMA,
    ],
)(_sc_l2_body)


def _sc_l3_body(p_hbm, src_hbm, dst_hbm, zacc_hbm,
                agg_out,
                srcv, dstv, rows, acc, sem):
  c = lax.axis_index("c")
  s = lax.axis_index("s")
  r0 = s * RT
  pltpu.sync_copy(zacc_hbm.at[pl.ds(r0, RT), :], acc.at[pl.ds(r0, RT), :])
  plsc.subcore_barrier()
  e0 = (c * 16 + s) * (EP // 32)

  def body(j, carry):
    off = e0 + j * B
    pltpu.sync_copy(src_hbm.at[pl.ds(off, B)], srcv)
    pltpu.sync_copy(dst_hbm.at[pl.ds(off, B)], dstv)
    pltpu.async_copy(p_hbm.at[srcv], rows, sem).wait()
    pltpu.sync_copy(rows, acc.at[dstv], add=True)
    return carry

  lax.fori_loop(0, (EP // 32) // B, body, 0)
  plsc.subcore_barrier()
  pltpu.sync_copy(acc.at[pl.ds(r0, RT), :], agg_out.at[c, pl.ds(r0, RT), :])


_sc_l3 = functools.partial(
    pl.kernel,
    out_type=jax.ShapeDtypeStruct((2, NP, 128), jnp.float32),
    mesh=_mesh,
    scratch_types=[
        pltpu.VMEM((B,), jnp.int32),
        pltpu.VMEM((B,), jnp.int32),
        pltpu.VMEM((B, 128), jnp.float32),
        pltpu.VMEM_SHARED((NP, 128), jnp.float32),
        pltpu.SemaphoreType.DMA,
    ],
)(_sc_l3_body)


MB = 512
MB3 = 1024


def _tc1_body(aggp, degp, x, wl, wr, b1, o):
  d = jnp.maximum(degp[0] + degp[1], 1.0)
  a = (aggp[0] + aggp[1]) / d
  h = jnp.dot(a, wl[...], preferred_element_type=jnp.float32)
  h = h + jnp.dot(x[...], wr[...], preferred_element_type=jnp.float32)
  o[...] = jnp.maximum(h + b1[...], 0.0)


def _tc2_body(agg4, degp, h1, wl4, wr, b2, wcat, h2o, pro):
  d = jnp.maximum(degp[0] + degp[1], 1.0)
  acc = jnp.dot(h1[...], wr[...], preferred_element_type=jnp.float32)
  for b in range(4):
    acc = acc + jnp.dot(agg4[b] / d, wl4[b],
                        preferred_element_type=jnp.float32)
  h2 = jnp.maximum(acc + b2[...], 0.0)
  h2o[...] = h2
  pro[...] = jnp.dot(h2, wcat[...], preferred_element_type=jnp.float32)


def _tc3_body(a3p, degp, prj, b3, o):
  d = jnp.maximum(degp[0] + degp[1], 1.0)
  sm = (a3p[0, :, 0:2] + a3p[1, :, 0:2]) / d
  o[...] = jnp.maximum(sm + prj[:, 2:4] + b3[...], 0.0)


@jax.jit
def kernel(x, edge_index, batch, W1_l, W1_r, b1, W2_l, W2_r, b2,
           W3_l, W3_r, b3):
  del batch
  f32 = jnp.float32
  x_p = jnp.pad(x, ((0, NP - N), (0, 0)))
  src_p = jnp.pad(edge_index[0], (0, EP - E))
  dst_p = jnp.pad(edge_index[1], (0, EP - E), constant_values=N)
  zacc = jnp.zeros((NP, 128), f32)
  zdeg = jnp.zeros((NP,), f32)
  wcat = jnp.concatenate([W3_l, W3_r, jnp.zeros((1024, 124), f32)], axis=1)

  agg1p, degp = _sc_l1(x_p, src_p, dst_p, zacc, zdeg)
  degp3 = degp.reshape(2, NP, 1)

  h1 = pl.pallas_call(
      _tc1_body,
      grid=(NP // MB,),
      in_specs=[
          pl.BlockSpec((2, MB, 128), lambda i: (0, i, 0)),
          pl.BlockSpec((2, MB, 1), lambda i: (0, i, 0)),
          pl.BlockSpec((MB, 128), lambda i: (i, 0)),
          pl.BlockSpec((128, 512), lambda i: (0, 0)),
          pl.BlockSpec((128, 512), lambda i: (0, 0)),
          pl.BlockSpec((1, 512), lambda i: (0, 0)),
      ],
      out_specs=pl.BlockSpec((MB, 512), lambda i: (i, 0)),
      out_shape=jax.ShapeDtypeStruct((NP, 512), f32),
  )(agg1p, degp3, x_p, W1_l, W1_r, b1.reshape(1, 512))

  agg2 = _sc_l2(h1.reshape(NP * 4, 128), src_p, dst_p, zacc)

  h2, pr = pl.pallas_call(
      _tc2_body,
      grid=(NP // MB,),
      in_specs=[
          pl.BlockSpec((4, MB, 128), lambda i: (0, i, 0)),
          pl.BlockSpec((2, MB, 1), lambda i: (0, i, 0)),
          pl.BlockSpec((MB, 512), lambda i: (i, 0)),
          pl.BlockSpec((4, 128, 1024), lambda i: (0, 0, 0)),
          pl.BlockSpec((512, 1024), lambda i: (0, 0)),
          pl.BlockSpec((1, 1024), lambda i: (0, 0)),
          pl.BlockSpec((1024, 128), lambda i: (0, 0)),
      ],
      out_specs=[
          pl.BlockSpec((MB, 1024), lambda i: (i, 0)),
          pl.BlockSpec((MB, 128), lambda i: (i, 0)),
      ],
      out_shape=[
          jax.ShapeDtypeStruct((NP, 1024), f32),
          jax.ShapeDtypeStruct((NP, 128), f32),
      ],
  )(agg2, degp3, h1, W2_l.reshape(4, 128, 1024), W2_r,
    b2.reshape(1, 1024), wcat)
  del h2

  agg3p = _sc_l3(pr, src_p, dst_p, zacc)

  out = pl.pallas_call(
      _tc3_body,
      grid=(NP // MB3,),
      in_specs=[
          pl.BlockSpec((2, MB3, 128), lambda i: (0, i, 0)),
          pl.BlockSpec((2, MB3, 1), lambda i: (0, i, 0)),
          pl.BlockSpec((MB3, 128), lambda i: (i, 0)),
          pl.BlockSpec((1, 2), lambda i: (0, 0)),
      ],
      out_specs=pl.BlockSpec((MB3, 2), lambda i: (i, 0)),
      out_shape=jax.ShapeDtypeStruct((NP, 2), f32),
  )(agg3p, degp3, pr, b3.reshape(1, 2))

  return out[:N]


# speedup vs baseline: 4.7251x; 1.3033x over previous
import functools

import jax
import jax.numpy as jnp
from jax import lax
from jax.experimental import pallas as pl
from jax.experimental.pallas import tpu as pltpu
from jax.experimental.pallas import tpu_sc as plsc

N = 10000
NP = 10240
E = 160000
EP = 163840
EP2 = EP + 1024
B = 128
RT = NP // 16
NB1 = EP // 32 // B
NB2 = EP // 16 // B

_mesh = plsc.VectorSubcoreMesh(core_axis_name="c", subcore_axis_name="s")


def _sc_l1_body(x_hbm, src2_hbm, dst2_hbm, zacc_hbm, zdeg_hbm,
                agg_out, deg_out,
                srcall, dstall, r0buf, r1buf, ones, acc, dacc,
                sem0, sem1):
  c = lax.axis_index("c")
  s = lax.axis_index("s")
  row0 = s * RT
  pltpu.sync_copy(zacc_hbm.at[pl.ds(row0, RT), :], acc.at[pl.ds(row0, RT), :])
  pltpu.sync_copy(zdeg_hbm.at[pl.ds(row0, RT)], dacc.at[pl.ds(row0, RT)])
  for i in range(B // 16):
    ones[pl.ds(i * 16, 16)] = jnp.full((16,), 1.0, jnp.float32)
  bb0 = (c * 16 + s) * NB1
  pltpu.sync_copy(src2_hbm.at[pl.ds(bb0, NB1 + 8), :], srcall)
  pltpu.sync_copy(dst2_hbm.at[pl.ds(bb0, NB1), :], dstall)
  plsc.subcore_barrier()

  pltpu.async_copy(x_hbm.at[srcall.at[0]], r0buf, sem0)

  def body(jj, carry):
    j0 = 2 * jj
    pltpu.async_copy(x_hbm.at[srcall.at[j0 + 1]], r1buf, sem1)
    pltpu.make_async_copy(x_hbm.at[srcall.at[j0]], r0buf, sem0).wait()
    pltpu.sync_copy(r0buf, acc.at[dstall.at[j0]], add=True)
    pltpu.sync_copy(ones, dacc.at[dstall.at[j0]], add=True)
    pltpu.async_copy(x_hbm.at[srcall.at[j0 + 2]], r0buf, sem0)
    pltpu.make_async_copy(x_hbm.at[srcall.at[j0 + 1]], r1buf, sem1).wait()
    pltpu.sync_copy(r1buf, acc.at[dstall.at[j0 + 1]], add=True)
    pltpu.sync_copy(ones, dacc.at[dstall.at[j0 + 1]], add=True)
    return carry

  lax.fori_loop(0, NB1 // 2, body, 0)
  pltpu.make_async_copy(x_hbm.at[srcall.at[0]], r0buf, sem0).wait()
  plsc.subcore_barrier()
  pltpu.sync_copy(acc.at[pl.ds(row0, RT), :], agg_out.at[c, pl.ds(row0, RT), :])
  pltpu.sync_copy(dacc.at[pl.ds(row0, RT)], deg_out.at[c, pl.ds(row0, RT)])


_sc_l1 = functools.partial(
    pl.kernel,
    out_type=(jax.ShapeDtypeStruct((2, NP, 128), jnp.float32),
              jax.ShapeDtypeStruct((2, NP), jnp.float32)),
    mesh=_mesh,
    scratch_types=[
        pltpu.VMEM((NB1 + 8, B), jnp.int32),
        pltpu.VMEM((NB1, B), jnp.int32),
        pltpu.VMEM((B, 128), jnp.float32),
        pltpu.VMEM((B, 128), jnp.float32),
        pltpu.VMEM((B,), jnp.float32),
        pltpu.VMEM_SHARED((NP, 128), jnp.float32),
        pltpu.VMEM_SHARED((NP,), jnp.float32),
        pltpu.SemaphoreType.DMA,
        pltpu.SemaphoreType.DMA,
    ],
)(_sc_l1_body)


def _sc_l2_body(h14_hbm, src2_hbm, dst2_hbm, zacc_hbm,
                agg_out,
                idxall, dstall, r0buf, r1buf, acc,
                sem0, sem1):
  c = lax.axis_index("c")
  s = lax.axis_index("s")
  row0 = s * RT
  for r in range(2):
    fb = c * 2 + r
    pltpu.sync_copy(zacc_hbm.at[pl.ds(row0, RT), :], acc.at[pl.ds(row0, RT), :])
    plsc.subcore_barrier()
    for half in range(2):
      bb0 = s * NB2 + half * NB1
      pltpu.sync_copy(src2_hbm.at[pl.ds(bb0, NB1 + 8), :], idxall)
      pltpu.sync_copy(dst2_hbm.at[pl.ds(bb0, NB1), :], dstall)

      def idxbody(jj, carry):
        for i in range(B // 16):
          sl = pl.ds(i * 16, 16)
          idxall[jj, sl] = idxall[jj, sl] * 4 + fb
        return carry

      lax.fori_loop(0, NB1 + 8, idxbody, 0)

      pltpu.async_copy(h14_hbm.at[idxall.at[0]], r0buf, sem0)

      def body(jj, carry):
        j0 = 2 * jj
        pltpu.async_copy(h14_hbm.at[idxall.at[j0 + 1]], r1buf, sem1)
        pltpu.make_async_copy(h14_hbm.at[idxall.at[j0]], r0buf, sem0).wait()
        pltpu.sync_copy(r0buf, acc.at[dstall.at[j0]], add=True)
        pltpu.async_copy(h14_hbm.at[idxall.at[j0 + 2]], r0buf, sem0)
        pltpu.make_async_copy(h14_hbm.at[idxall.at[j0 + 1]], r1buf, sem1).wait()
        pltpu.sync_copy(r1buf, acc.at[dstall.at[j0 + 1]], add=True)
        return carry

      lax.fori_loop(0, NB1 // 2, body, 0)
      pltpu.make_async_copy(h14_hbm.at[idxall.at[0]], r0buf, sem0).wait()
    plsc.subcore_barrier()
    pltpu.sync_copy(acc.at[pl.ds(row0, RT), :],
                    agg_out.at[fb, pl.ds(row0, RT), :])
    plsc.subcore_barrier()


_sc_l2 = functools.partial(
    pl.kernel,
    out_type=jax.ShapeDtypeStruct((4, NP, 128), jnp.float32),
    mesh=_mesh,
    scratch_types=[
        pltpu.VMEM((NB1 + 8, B), jnp.int32),
        pltpu.VMEM((NB1, B), jnp.int32),
        pltpu.VMEM((B, 128), jnp.float32),
        pltpu.VMEM((B, 128), jnp.float32),
        pltpu.VMEM_SHARED((NP, 128), jnp.float32),
        pltpu.SemaphoreType.DMA,
        pltpu.SemaphoreType.DMA,
    ],
)(_sc_l2_body)


def _sc_l3_body(p_hbm, src2_hbm, dst2_hbm, zacc_hbm,
                agg_out,
                srcall, dstall, r0buf, r1buf, acc,
                sem0, sem1):
  c = lax.axis_index("c")
  s = lax.axis_index("s")
  row0 = s * RT
  pltpu.sync_copy(zacc_hbm.at[pl.ds(row0, RT), :], acc.at[pl.ds(row0, RT), :])
  bb0 = (c * 16 + s) * NB1
  pltpu.sync_copy(src2_hbm.at[pl.ds(bb0, NB1 + 8), :], srcall)
  pltpu.sync_copy(dst2_hbm.at[pl.ds(bb0, NB1), :], dstall)
  plsc.subcore_barrier()

  pltpu.async_copy(p_hbm.at[srcall.at[0]], r0buf, sem0)

  def body(jj, carry):
    j0 = 2 * jj
    pltpu.async_copy(p_hbm.at[srcall.at[j0 + 1]], r1buf, sem1)
    pltpu.make_async_copy(p_hbm.at[srcall.at[j0]], r0buf, sem0).wait()
    pltpu.sync_copy(r0buf, acc.at[dstall.at[j0]], add=True)
    pltpu.async_copy(p_hbm.at[srcall.at[j0 + 2]], r0buf, sem0)
    pltpu.make_async_copy(p_hbm.at[srcall.at[j0 + 1]], r1buf, sem1).wait()
    pltpu.sync_copy(r1buf, acc.at[dstall.at[j0 + 1]], add=True)
    return carry

  lax.fori_loop(0, NB1 // 2, body, 0)
  pltpu.make_async_copy(p_hbm.at[srcall.at[0]], r0buf, sem0).wait()
  plsc.subcore_barrier()
  pltpu.sync_copy(acc.at[pl.ds(row0, RT), :], agg_out.at[c, pl.ds(row0, RT), :])


_sc_l3 = functools.partial(
    pl.kernel,
    out_type=jax.ShapeDtypeStruct((2, NP, 128), jnp.float32),
    mesh=_mesh,
    scratch_types=[
        pltpu.VMEM((NB1 + 8, B), jnp.int32),
        pltpu.VMEM((NB1, B), jnp.int32),
        pltpu.VMEM((B, 128), jnp.float32),
        pltpu.VMEM((B, 128), jnp.float32),
        pltpu.VMEM_SHARED((NP, 128), jnp.float32),
        pltpu.SemaphoreType.DMA,
        pltpu.SemaphoreType.DMA,
    ],
)(_sc_l3_body)


MB = 512
MB3 = 1024


def _tc1_body(aggp, degp, x, wl, wr, b1, o):
  d = jnp.maximum(degp[0] + degp[1], 1.0)
  a = (aggp[0] + aggp[1]) / d
  h = jnp.dot(a, wl[...], preferred_element_type=jnp.float32)
  h = h + jnp.dot(x[...], wr[...], preferred_element_type=jnp.float32)
  o[...] = jnp.maximum(h + b1[...], 0.0)


def _tc2_body(agg4, degp, h1, wl4, wr, b2, wcat, h2o, pro):
  d = jnp.maximum(degp[0] + degp[1], 1.0)
  acc = jnp.dot(h1[...], wr[...], preferred_element_type=jnp.float32)
  for b in range(4):
    acc = acc + jnp.dot(agg4[b] / d, wl4[b],
                        preferred_element_type=jnp.float32)
  h2 = jnp.maximum(acc + b2[...], 0.0)
  h2o[...] = h2
  pro[...] = jnp.dot(h2, wcat[...], preferred_element_type=jnp.float32)


def _tc3_body(a3p, degp, prj, b3, o):
  d = jnp.maximum(degp[0] + degp[1], 1.0)
  sm = (a3p[0, :, 0:2] + a3p[1, :, 0:2]) / d
  o[...] = jnp.maximum(sm + prj[:, 2:4] + b3[...], 0.0)


@jax.jit
def kernel(x, edge_index, batch, W1_l, W1_r, b1, W2_l, W2_r, b2,
           W3_l, W3_r, b3):
  del batch
  f32 = jnp.float32
  x_p = jnp.pad(x, ((0, NP - N), (0, 0)))
  src2 = jnp.pad(edge_index[0], (0, EP2 - E)).reshape(EP2 // B, B)
  dst2 = jnp.pad(edge_index[1], (0, EP - E),
                 constant_values=N).reshape(EP // B, B)
  zacc = jnp.zeros((NP, 128), f32)
  zdeg = jnp.zeros((NP,), f32)
  wcat = jnp.concatenate([W3_l, W3_r, jnp.zeros((1024, 124), f32)], axis=1)

  agg1p, degp = _sc_l1(x_p, src2, dst2, zacc, zdeg)
  degp3 = degp.reshape(2, NP, 1)

  h1 = pl.pallas_call(
      _tc1_body,
      grid=(NP // MB,),
      in_specs=[
          pl.BlockSpec((2, MB, 128), lambda i: (0, i, 0)),
          pl.BlockSpec((2, MB, 1), lambda i: (0, i, 0)),
          pl.BlockSpec((MB, 128), lambda i: (i, 0)),
          pl.BlockSpec((128, 512), lambda i: (0, 0)),
          pl.BlockSpec((128, 512), lambda i: (0, 0)),
          pl.BlockSpec((1, 512), lambda i: (0, 0)),
      ],
      out_specs=pl.BlockSpec((MB, 512), lambda i: (i, 0)),
      out_shape=jax.ShapeDtypeStruct((NP, 512), f32),
  )(agg1p, degp3, x_p, W1_l, W1_r, b1.reshape(1, 512))

  agg2 = _sc_l2(h1.reshape(NP * 4, 128), src2, dst2, zacc)

  h2, pr = pl.pallas_call(
      _tc2_body,
      grid=(NP // MB,),
      in_specs=[
          pl.BlockSpec((4, MB, 128), lambda i: (0, i, 0)),
          pl.BlockSpec((2, MB, 1), lambda i: (0, i, 0)),
          pl.BlockSpec((MB, 512), lambda i: (i, 0)),
          pl.BlockSpec((4, 128, 1024), lambda i: (0, 0, 0)),
          pl.BlockSpec((512, 1024), lambda i: (0, 0)),
          pl.BlockSpec((1, 1024), lambda i: (0, 0)),
          pl.BlockSpec((1024, 128), lambda i: (0, 0)),
      ],
      out_specs=[
          pl.BlockSpec((MB, 1024), lambda i: (i, 0)),
          pl.BlockSpec((MB, 128), lambda i: (i, 0)),
      ],
      out_shape=[
          jax.ShapeDtypeStruct((NP, 1024), f32),
          jax.ShapeDtypeStruct((NP, 128), f32),
      ],
  )(agg2, degp3, h1, W2_l.reshape(4, 128, 1024), W2_r,
    b2.reshape(1, 1024), wcat)
  del h2

  agg3p = _sc_l3(pr, src2, dst2, zacc)

  out = pl.pallas_call(
      _tc3_body,
      grid=(NP // MB3,),
      in_specs=[
          pl.BlockSpec((2, MB3, 128), lambda i: (0, i, 0)),
          pl.BlockSpec((2, MB3, 1), lambda i: (0, i, 0)),
          pl.BlockSpec((MB3, 128), lambda i: (i, 0)),
          pl.BlockSpec((1, 2), lambda i: (0, 0)),
      ],
      out_specs=pl.BlockSpec((MB3, 2), lambda i: (i, 0)),
      out_shape=jax.ShapeDtypeStruct((NP, 2), f32),
  )(agg3p, degp3, pr, b3.reshape(1, 2))

  return out[:N]


# speedup vs baseline: 4.7304x; 1.0011x over previous
import functools

import jax
import jax.numpy as jnp
from jax import lax
from jax.experimental import pallas as pl
from jax.experimental.pallas import tpu as pltpu
from jax.experimental.pallas import tpu_sc as plsc

N = 10000
NP = 10240
E = 160000
EP = 163840
EP2 = EP + 1024
B = 128
RT = NP // 16
NB1 = EP // 32 // B
NB2 = EP // 16 // B

_mesh = plsc.VectorSubcoreMesh(core_axis_name="c", subcore_axis_name="s")


def _sc_l1_body(x_hbm, src2_hbm, dst2_hbm, zacc_hbm, zdeg_hbm,
                agg_out, deg_out,
                srcall, dstall, r0buf, r1buf, ones, acc, dacc,
                sem0, sem1):
  c = lax.axis_index("c")
  s = lax.axis_index("s")
  row0 = s * RT
  pltpu.sync_copy(zacc_hbm.at[pl.ds(row0, RT), :], acc.at[pl.ds(row0, RT), :])
  pltpu.sync_copy(zdeg_hbm.at[pl.ds(row0, RT)], dacc.at[pl.ds(row0, RT)])
  for i in range(B // 16):
    ones[pl.ds(i * 16, 16)] = jnp.full((16,), 1.0, jnp.float32)
  bb0 = (c * 16 + s) * NB1
  pltpu.sync_copy(src2_hbm.at[pl.ds(bb0, NB1 + 8), :], srcall)
  pltpu.sync_copy(dst2_hbm.at[pl.ds(bb0, NB1), :], dstall)
  plsc.subcore_barrier()

  pltpu.async_copy(x_hbm.at[srcall.at[0]], r0buf, sem0)

  def body(jj, carry):
    j0 = 2 * jj
    pltpu.async_copy(x_hbm.at[srcall.at[j0 + 1]], r1buf, sem1)
    pltpu.make_async_copy(x_hbm.at[srcall.at[j0]], r0buf, sem0).wait()
    pltpu.sync_copy(r0buf, acc.at[dstall.at[j0]], add=True)
    pltpu.sync_copy(ones, dacc.at[dstall.at[j0]], add=True)
    pltpu.async_copy(x_hbm.at[srcall.at[j0 + 2]], r0buf, sem0)
    pltpu.make_async_copy(x_hbm.at[srcall.at[j0 + 1]], r1buf, sem1).wait()
    pltpu.sync_copy(r1buf, acc.at[dstall.at[j0 + 1]], add=True)
    pltpu.sync_copy(ones, dacc.at[dstall.at[j0 + 1]], add=True)
    return carry

  lax.fori_loop(0, NB1 // 2, body, 0)
  pltpu.make_async_copy(x_hbm.at[srcall.at[0]], r0buf, sem0).wait()
  plsc.subcore_barrier()
  pltpu.sync_copy(acc.at[pl.ds(row0, RT), :], agg_out.at[c, pl.ds(row0, RT), :])
  pltpu.sync_copy(dacc.at[pl.ds(row0, RT)], deg_out.at[c, pl.ds(row0, RT)])


_sc_l1 = functools.partial(
    pl.kernel,
    out_type=(jax.ShapeDtypeStruct((2, NP, 128), jnp.float32),
              jax.ShapeDtypeStruct((2, NP), jnp.float32)),
    mesh=_mesh,
    scratch_types=[
        pltpu.VMEM((NB1 + 8, B), jnp.int32),
        pltpu.VMEM((NB1, B), jnp.int32),
        pltpu.VMEM((B, 128), jnp.float32),
        pltpu.VMEM((B, 128), jnp.float32),
        pltpu.VMEM((B,), jnp.float32),
        pltpu.VMEM_SHARED((NP, 128), jnp.float32),
        pltpu.VMEM_SHARED((NP,), jnp.float32),
        pltpu.SemaphoreType.DMA,
        pltpu.SemaphoreType.DMA,
    ],
)(_sc_l1_body)


def _sc_l2_body(h14_hbm, src2_hbm, dst2_hbm, zacc_hbm,
                agg_out,
                idxall, dstall, r0buf, r1buf, acc,
                sem0, sem1):
  c = lax.axis_index("c")
  s = lax.axis_index("s")
  row0 = s * RT
  for r in range(2):
    fb = c * 2 + r
    pltpu.sync_copy(zacc_hbm.at[pl.ds(row0, RT), :], acc.at[pl.ds(row0, RT), :])
    plsc.subcore_barrier()
    for half in range(2):
      bb0 = s * NB2 + half * NB1
      pltpu.sync_copy(src2_hbm.at[pl.ds(bb0, NB1 + 8), :], idxall)
      pltpu.sync_copy(dst2_hbm.at[pl.ds(bb0, NB1), :], dstall)

      def idxbody(jj, carry):
        for i in range(B // 16):
          sl = pl.ds(i * 16, 16)
          idxall[jj, sl] = idxall[jj, sl] * 4 + fb
        return carry

      lax.fori_loop(0, NB1 + 8, idxbody, 0)

      pltpu.async_copy(h14_hbm.at[idxall.at[0]], r0buf, sem0)

      def body(jj, carry):
        j0 = 2 * jj
        pltpu.async_copy(h14_hbm.at[idxall.at[j0 + 1]], r1buf, sem1)
        pltpu.make_async_copy(h14_hbm.at[idxall.at[j0]], r0buf, sem0).wait()
        pltpu.sync_copy(r0buf, acc.at[dstall.at[j0]], add=True)
        pltpu.async_copy(h14_hbm.at[idxall.at[j0 + 2]], r0buf, sem0)
        pltpu.make_async_copy(h14_hbm.at[idxall.at[j0 + 1]], r1buf, sem1).wait()
        pltpu.sync_copy(r1buf, acc.at[dstall.at[j0 + 1]], add=True)
        return carry

      lax.fori_loop(0, NB1 // 2, body, 0)
      pltpu.make_async_copy(h14_hbm.at[idxall.at[0]], r0buf, sem0).wait()
    plsc.subcore_barrier()
    pltpu.sync_copy(acc.at[pl.ds(row0, RT), :],
                    agg_out.at[fb, pl.ds(row0, RT), :])
    plsc.subcore_barrier()


_sc_l2 = functools.partial(
    pl.kernel,
    out_type=jax.ShapeDtypeStruct((4, NP, 128), jnp.float32),
    mesh=_mesh,
    scratch_types=[
        pltpu.VMEM((NB1 + 8, B), jnp.int32),
        pltpu.VMEM((NB1, B), jnp.int32),
        pltpu.VMEM((B, 128), jnp.float32),
        pltpu.VMEM((B, 128), jnp.float32),
        pltpu.VMEM_SHARED((NP, 128), jnp.float32),
        pltpu.SemaphoreType.DMA,
        pltpu.SemaphoreType.DMA,
    ],
)(_sc_l2_body)


def _sc_l3_body(p_hbm, src2_hbm, dst2_hbm, zacc_hbm,
                agg_out,
                srcall, dstall, r0buf, r1buf, acc,
                sem0, sem1):
  c = lax.axis_index("c")
  s = lax.axis_index("s")
  row0 = s * RT
  pltpu.sync_copy(zacc_hbm.at[pl.ds(row0, RT), :], acc.at[pl.ds(row0, RT), :])
  bb0 = (c * 16 + s) * NB1
  pltpu.sync_copy(src2_hbm.at[pl.ds(bb0, NB1 + 8), :], srcall)
  pltpu.sync_copy(dst2_hbm.at[pl.ds(bb0, NB1), :], dstall)
  plsc.subcore_barrier()

  pltpu.async_copy(p_hbm.at[srcall.at[0]], r0buf, sem0)

  def body(jj, carry):
    j0 = 2 * jj
    pltpu.async_copy(p_hbm.at[srcall.at[j0 + 1]], r1buf, sem1)
    pltpu.make_async_copy(p_hbm.at[srcall.at[j0]], r0buf, sem0).wait()
    pltpu.sync_copy(r0buf, acc.at[dstall.at[j0]], add=True)
    pltpu.async_copy(p_hbm.at[srcall.at[j0 + 2]], r0buf, sem0)
    pltpu.make_async_copy(p_hbm.at[srcall.at[j0 + 1]], r1buf, sem1).wait()
    pltpu.sync_copy(r1buf, acc.at[dstall.at[j0 + 1]], add=True)
    return carry

  lax.fori_loop(0, NB1 // 2, body, 0)
  pltpu.make_async_copy(p_hbm.at[srcall.at[0]], r0buf, sem0).wait()
  plsc.subcore_barrier()
  pltpu.sync_copy(acc.at[pl.ds(row0, RT), :], agg_out.at[c, pl.ds(row0, RT), :])


_sc_l3 = functools.partial(
    pl.kernel,
    out_type=jax.ShapeDtypeStruct((2, NP, 128), jnp.float32),
    mesh=_mesh,
    scratch_types=[
        pltpu.VMEM((NB1 + 8, B), jnp.int32),
        pltpu.VMEM((NB1, B), jnp.int32),
        pltpu.VMEM((B, 128), jnp.float32),
        pltpu.VMEM((B, 128), jnp.float32),
        pltpu.VMEM_SHARED((NP, 128), jnp.float32),
        pltpu.SemaphoreType.DMA,
        pltpu.SemaphoreType.DMA,
    ],
)(_sc_l3_body)


MB = 512
MB3 = 1024


def _tc1_body(aggp, degp, x, wl, wr, b1, o):
  d = jnp.maximum(degp[0] + degp[1], 1.0)
  a = (aggp[0] + aggp[1]) / d
  h = jnp.dot(a, wl[...], preferred_element_type=jnp.float32)
  h = h + jnp.dot(x[...], wr[...], preferred_element_type=jnp.float32)
  o[...] = jnp.maximum(h + b1[...], 0.0)


def _tc2_body(agg4, degp, h1, wl4, wr, b2, wcat, h2o, pro):
  d = jnp.maximum(degp[0] + degp[1], 1.0)
  acc = jnp.dot(h1[...], wr[...], preferred_element_type=jnp.float32)
  for b in range(4):
    acc = acc + jnp.dot(agg4[b] / d, wl4[b],
                        preferred_element_type=jnp.float32)
  h2 = jnp.maximum(acc + b2[...], 0.0)
  h2o[...] = h2
  pro[...] = jnp.dot(h2, wcat[...], preferred_element_type=jnp.float32)


def _tc3_body(a3p, degp, prj, b3, o):
  d = jnp.maximum(degp[0] + degp[1], 1.0)
  sm = (a3p[0, :, 0:2] + a3p[1, :, 0:2]) / d
  o[...] = jnp.maximum(sm + prj[:, 2:4] + b3[...], 0.0)


@jax.jit
def kernel(x, edge_index, batch, W1_l, W1_r, b1, W2_l, W2_r, b2,
           W3_l, W3_r, b3):
  del batch
  f32 = jnp.float32
  x_p = jnp.pad(x, ((0, NP - N), (0, 0)))
  src2 = jnp.pad(edge_index[0], (0, EP2 - E)).reshape(EP2 // B, B)
  pad_dst = N + jnp.arange(EP - E, dtype=jnp.int32) % (NP - N)
  dst2 = jnp.concatenate([edge_index[1], pad_dst]).reshape(EP // B, B)
  zacc = jnp.zeros((NP, 128), f32)
  zdeg = jnp.zeros((NP,), f32)
  wcat = jnp.concatenate([W3_l, W3_r, jnp.zeros((1024, 124), f32)], axis=1)

  agg1p, degp = _sc_l1(x_p, src2, dst2, zacc, zdeg)
  degp3 = degp.reshape(2, NP, 1)

  h1 = pl.pallas_call(
      _tc1_body,
      grid=(NP // MB,),
      in_specs=[
          pl.BlockSpec((2, MB, 128), lambda i: (0, i, 0)),
          pl.BlockSpec((2, MB, 1), lambda i: (0, i, 0)),
          pl.BlockSpec((MB, 128), lambda i: (i, 0)),
          pl.BlockSpec((128, 512), lambda i: (0, 0)),
          pl.BlockSpec((128, 512), lambda i: (0, 0)),
          pl.BlockSpec((1, 512), lambda i: (0, 0)),
      ],
      out_specs=pl.BlockSpec((MB, 512), lambda i: (i, 0)),
      out_shape=jax.ShapeDtypeStruct((NP, 512), f32),
  )(agg1p, degp3, x_p, W1_l, W1_r, b1.reshape(1, 512))

  agg2 = _sc_l2(h1.reshape(NP * 4, 128), src2, dst2, zacc)

  h2, pr = pl.pallas_call(
      _tc2_body,
      grid=(NP // MB,),
      in_specs=[
          pl.BlockSpec((4, MB, 128), lambda i: (0, i, 0)),
          pl.BlockSpec((2, MB, 1), lambda i: (0, i, 0)),
          pl.BlockSpec((MB, 512), lambda i: (i, 0)),
          pl.BlockSpec((4, 128, 1024), lambda i: (0, 0, 0)),
          pl.BlockSpec((512, 1024), lambda i: (0, 0)),
          pl.BlockSpec((1, 1024), lambda i: (0, 0)),
          pl.BlockSpec((1024, 128), lambda i: (0, 0)),
      ],
      out_specs=[
          pl.BlockSpec((MB, 1024), lambda i: (i, 0)),
          pl.BlockSpec((MB, 128), lambda i: (i, 0)),
      ],
      out_shape=[
          jax.ShapeDtypeStruct((NP, 1024), f32),
          jax.ShapeDtypeStruct((NP, 128), f32),
      ],
  )(agg2, degp3, h1, W2_l.reshape(4, 128, 1024), W2_r,
    b2.reshape(1, 1024), wcat)
  del h2

  agg3p = _sc_l3(pr, src2, dst2, zacc)

  out = pl.pallas_call(
      _tc3_body,
      grid=(NP // MB3,),
      in_specs=[
          pl.BlockSpec((2, MB3, 128), lambda i: (0, i, 0)),
          pl.BlockSpec((2, MB3, 1), lambda i: (0, i, 0)),
          pl.BlockSpec((MB3, 128), lambda i: (i, 0)),
          pl.BlockSpec((1, 2), lambda i: (0, 0)),
      ],
      out_specs=pl.BlockSpec((MB3, 2), lambda i: (i, 0)),
      out_shape=jax.ShapeDtypeStruct((NP, 2), f32),
  )(agg3p, degp3, pr, b3.reshape(1, 2))

  return out[:N]
